# baseline (speedup 1.0000x reference)
import functools

import jax
import jax.numpy as jnp
from jax import lax
from jax.experimental import pallas as pl
from jax.experimental.pallas import tpu as pltpu
from jax.experimental.pallas import tpu_sc as plsc

NC = 2
NS = 16
L = 16
NW = NC * NS

B = 16384
D = 128
BPW = B // NW
CH = 128
NCHUNK = BPW // CH


def _sc_body(user_ref, news_ref, ut_ref, nt_ref, out_ref,
             uidx, nidx, ubuf0, ubuf1, nbuf0, nbuf1, scores, usem, nsem):
    ubufs = (ubuf0, ubuf1)
    nbufs = (nbuf0, nbuf1)
    wid = lax.axis_index("s") * NC + lax.axis_index("c")
    base = wid * BPW

    for c in range(NCHUNK):
        pltpu.sync_copy(user_ref.at[pl.ds(base + c * CH, CH)], uidx.at[c])
        pltpu.sync_copy(news_ref.at[pl.ds(base + c * CH, CH)], nidx.at[c])

    uh = [None] * NCHUNK
    nh = [None] * NCHUNK
    uh[0] = pltpu.async_copy(ut_ref.at[uidx.at[0]], ubufs[0], usem)
    nh[0] = pltpu.async_copy(nt_ref.at[nidx.at[0]], nbufs[0], nsem)

    for c in range(NCHUNK):
        cur = c % 2
        uh[c].wait()
        nh[c].wait()
        if c + 1 < NCHUNK:
            nxt = (c + 1) % 2
            uh[c + 1] = pltpu.async_copy(ut_ref.at[uidx.at[c + 1]], ubufs[nxt], usem)
            nh[c + 1] = pltpu.async_copy(nt_ref.at[nidx.at[c + 1]], nbufs[nxt], nsem)

        lanes = lax.iota(jnp.int32, L)

        def grp_body(g, _, cur=cur, c=c):
            vec = jnp.zeros((L,), jnp.float32)
            for r in range(L):
                i = g * L + r
                acc = ubufs[cur][i, pl.ds(0, L)] * nbufs[cur][i, pl.ds(0, L)]
                for j in range(1, D // L):
                    acc = acc + (ubufs[cur][i, pl.ds(j * L, L)]
                                 * nbufs[cur][i, pl.ds(j * L, L)])
                s = jnp.sum(acc)
                vec = jnp.where(lanes == r, s, vec)
            scores[pl.ds(c * CH + g * L, L)] = vec
            return 0

        lax.fori_loop(0, CH // L, grp_body, 0)

    pltpu.sync_copy(scores, out_ref.at[pl.ds(base, BPW)])


@jax.jit
def _scores(user, news, user_table, news_table):
    mesh = plsc.VectorSubcoreMesh(core_axis_name="c", subcore_axis_name="s",
                                  num_cores=NC, num_subcores=NS)
    call = functools.partial(
        pl.kernel,
        out_type=jax.ShapeDtypeStruct((B,), jnp.float32),
        mesh=mesh,
        compiler_params=pltpu.CompilerParams(needs_layout_passes=False),
        scratch_types=[
            pltpu.VMEM((NCHUNK, CH), jnp.int32),
            pltpu.VMEM((NCHUNK, CH), jnp.int32),
            pltpu.VMEM((CH, D), jnp.float32),
            pltpu.VMEM((CH, D), jnp.float32),
            pltpu.VMEM((CH, D), jnp.float32),
            pltpu.VMEM((CH, D), jnp.float32),
            pltpu.VMEM((BPW,), jnp.float32),
            pltpu.SemaphoreType.DMA,
            pltpu.SemaphoreType.DMA,
        ],
    )(_sc_body)
    return call(user.astype(jnp.int32), news.astype(jnp.int32),
                user_table, news_table)


def kernel(user, news, user_table, news_table):
    scores = _scores(user, news, user_table, news_table)
    return (user_table, news_table, scores)


# speedup vs baseline: 1.0001x; 1.0001x over previous
import functools

import jax
import jax.numpy as jnp
from jax import lax
from jax.experimental import pallas as pl
from jax.experimental.pallas import tpu as pltpu
from jax.experimental.pallas import tpu_sc as plsc

NC = 2
NS = 16
L = 16
NW = NC * NS

B = 16384
D = 128
BPW = B // NW
CH = 128
NCHUNK = BPW // CH


def _sc_body(user_ref, news_ref, ut_ref, nt_ref, out_ref,
             uidx, nidx, ubuf0, ubuf1, nbuf0, nbuf1, scores, usem, nsem):
    ubufs = (ubuf0, ubuf1)
    nbufs = (nbuf0, nbuf1)
    wid = lax.axis_index("s") * NC + lax.axis_index("c")
    base = wid * BPW

    for c in range(NCHUNK):
        pltpu.sync_copy(user_ref.at[pl.ds(base + c * CH, CH)], uidx.at[c])
        pltpu.sync_copy(news_ref.at[pl.ds(base + c * CH, CH)], nidx.at[c])

    uh = [None] * NCHUNK
    nh = [None] * NCHUNK
    uh[0] = pltpu.async_copy(ut_ref.at[uidx.at[0]], ubufs[0], usem)
    nh[0] = pltpu.async_copy(nt_ref.at[nidx.at[0]], nbufs[0], nsem)

    for c in range(NCHUNK):
        cur = c % 2
        uh[c].wait()
        nh[c].wait()
        if c + 1 < NCHUNK:
            nxt = (c + 1) % 2
            uh[c + 1] = pltpu.async_copy(ut_ref.at[uidx.at[c + 1]], ubufs[nxt], usem)
            nh[c + 1] = pltpu.async_copy(nt_ref.at[nidx.at[c + 1]], nbufs[nxt], nsem)

        lanes = lax.iota(jnp.int32, L)

        def grp_body(g, _, cur=cur, c=c):
            vec = jnp.zeros((L,), jnp.float32)
            for r in range(L):
                i = g * L + r
                acc = ubufs[cur][i, pl.ds(0, L)] * nbufs[cur][i, pl.ds(0, L)]
                for j in range(1, D // L):
                    acc = acc + (ubufs[cur][i, pl.ds(j * L, L)]
                                 * nbufs[cur][i, pl.ds(j * L, L)])
                s = jnp.sum(acc)
                vec = jnp.where(lanes == r, s, vec)
            scores[pl.ds(c * CH + g * L, L)] = vec
            return 0

        lax.fori_loop(0, CH // L, grp_body, 0)

    pltpu.sync_copy(scores, out_ref.at[pl.ds(base, BPW)])


@jax.jit
def _scores(user, news, user_table, news_table):
    mesh = plsc.VectorSubcoreMesh(core_axis_name="c", subcore_axis_name="s",
                                  num_cores=NC, num_subcores=NS)
    call = functools.partial(
        pl.kernel,
        out_type=jax.ShapeDtypeStruct((B,), jnp.float32),
        mesh=mesh,
        compiler_params=pltpu.CompilerParams(needs_layout_passes=False),
        scratch_types=[
            pltpu.VMEM((NCHUNK, CH), jnp.int32),
            pltpu.VMEM((NCHUNK, CH), jnp.int32),
            pltpu.VMEM((CH, D), jnp.float32),
            pltpu.VMEM((CH, D), jnp.float32),
            pltpu.VMEM((CH, D), jnp.float32),
            pltpu.VMEM((CH, D), jnp.float32),
            pltpu.VMEM((BPW,), jnp.float32),
            pltpu.SemaphoreType.DMA,
            pltpu.SemaphoreType.DMA,
        ],
    )(_sc_body)
    return call(user.astype(jnp.int32), news.astype(jnp.int32),
                user_table, news_table)


def kernel(user, news, user_table, news_table):
    scores = _scores(user, news, user_table, news_table)
    ut = jnp.copy(user_table)
    nt = jnp.copy(news_table)
    return (ut, nt, scores)


# speedup vs baseline: 1.0006x; 1.0005x over previous
import functools

import jax
import jax.numpy as jnp
from jax import lax
from jax.experimental import pallas as pl
from jax.experimental.pallas import tpu as pltpu
from jax.experimental.pallas import tpu_sc as plsc

NC = 2
NS = 16
L = 16
NW = NC * NS

B = 16384
D = 128
BPW = B // NW
CH = 128
NCHUNK = BPW // CH


def _sc_body(user_ref, news_ref, ut_ref, nt_ref, out_ref,
             uidx, nidx, ubuf0, ubuf1, nbuf0, nbuf1, scores, usem, nsem):
    ubufs = (ubuf0, ubuf1)
    nbufs = (nbuf0, nbuf1)
    wid = lax.axis_index("s") * NC + lax.axis_index("c")
    base = wid * BPW

    for c in range(NCHUNK):
        pltpu.sync_copy(user_ref.at[pl.ds(base + c * CH, CH)], uidx.at[c])
        pltpu.sync_copy(news_ref.at[pl.ds(base + c * CH, CH)], nidx.at[c])

    uh = [None] * NCHUNK
    nh = [None] * NCHUNK
    uh[0] = pltpu.async_copy(ut_ref.at[uidx.at[0]], ubufs[0], usem)
    nh[0] = pltpu.async_copy(nt_ref.at[nidx.at[0]], nbufs[0], nsem)

    for c in range(NCHUNK):
        cur = c % 2
        uh[c].wait()
        nh[c].wait()
        if c + 1 < NCHUNK:
            nxt = (c + 1) % 2
            uh[c + 1] = pltpu.async_copy(ut_ref.at[uidx.at[c + 1]], ubufs[nxt], usem)
            nh[c + 1] = pltpu.async_copy(nt_ref.at[nidx.at[c + 1]], nbufs[nxt], nsem)

        lanes = lax.iota(jnp.int32, L)

        def grp_body(g, _, cur=cur, c=c):
            vec = jnp.zeros((L,), jnp.float32)
            for r in range(L):
                i = g * L + r
                acc = ubufs[cur][i, pl.ds(0, L)] * nbufs[cur][i, pl.ds(0, L)]
                for j in range(1, D // L):
                    acc = acc + (ubufs[cur][i, pl.ds(j * L, L)]
                                 * nbufs[cur][i, pl.ds(j * L, L)])
                s = jnp.sum(acc)
                vec = jnp.where(lanes == r, s, vec)
            scores[pl.ds(c * CH + g * L, L)] = vec
            return 0

        lax.fori_loop(0, CH // L, grp_body, 0)

    pltpu.sync_copy(scores, out_ref.at[pl.ds(base, BPW)])


@jax.jit
def _scores(user, news, user_table, news_table):
    mesh = plsc.VectorSubcoreMesh(core_axis_name="c", subcore_axis_name="s",
                                  num_cores=NC, num_subcores=NS)
    call = functools.partial(
        pl.kernel,
        out_type=jax.ShapeDtypeStruct((B,), jnp.float32),
        mesh=mesh,
        compiler_params=pltpu.CompilerParams(needs_layout_passes=False),
        scratch_types=[
            pltpu.VMEM((NCHUNK, CH), jnp.int32),
            pltpu.VMEM((NCHUNK, CH), jnp.int32),
            pltpu.VMEM((CH, D), jnp.float32),
            pltpu.VMEM((CH, D), jnp.float32),
            pltpu.VMEM((CH, D), jnp.float32),
            pltpu.VMEM((CH, D), jnp.float32),
            pltpu.VMEM((BPW,), jnp.float32),
            pltpu.SemaphoreType.DMA,
            pltpu.SemaphoreType.DMA,
        ],
    )(_sc_body)
    return call(user.astype(jnp.int32), news.astype(jnp.int32),
                user_table, news_table)


def kernel(user, news, user_table, news_table):
    scores = _scores(user, news, user_table, news_table)
    one = (1 + 0 * jnp.min(news)).astype(jnp.float32)
    ut = user_table * one
    nt = news_table * one
    return (ut, nt, scores)


# speedup vs baseline: 1.0864x; 1.0858x over previous
import functools

import jax
import jax.numpy as jnp
from jax import lax
from jax.experimental import pallas as pl
from jax.experimental.pallas import tpu as pltpu
from jax.experimental.pallas import tpu_sc as plsc

NC = 2
NS = 16
L = 16
NW = NC * NS

B = 16384
D = 128
BPW = B // NW
CH = 128
NCHUNK = BPW // CH


def _sc_body(user_ref, news_ref, ut_ref, nt_ref, out_ref,
             uidx, nidx, ubuf0, ubuf1, nbuf0, nbuf1, scores, usem, nsem):
    ubufs = (ubuf0, ubuf1)
    nbufs = (nbuf0, nbuf1)
    wid = lax.axis_index("s") * NC + lax.axis_index("c")
    base = wid * BPW

    for c in range(NCHUNK):
        pltpu.sync_copy(user_ref.at[pl.ds(base + c * CH, CH)], uidx.at[c])
        pltpu.sync_copy(news_ref.at[pl.ds(base + c * CH, CH)], nidx.at[c])

    uh = [None] * NCHUNK
    nh = [None] * NCHUNK
    uh[0] = pltpu.async_copy(ut_ref.at[uidx.at[0]], ubufs[0], usem)
    nh[0] = pltpu.async_copy(nt_ref.at[nidx.at[0]], nbufs[0], nsem)

    for c in range(NCHUNK):
        cur = c % 2
        uh[c].wait()
        nh[c].wait()
        if c + 1 < NCHUNK:
            nxt = (c + 1) % 2
            uh[c + 1] = pltpu.async_copy(ut_ref.at[uidx.at[c + 1]], ubufs[nxt], usem)
            nh[c + 1] = pltpu.async_copy(nt_ref.at[nidx.at[c + 1]], nbufs[nxt], nsem)

        lanes = lax.iota(jnp.int32, L)

        def grp_body(g, _, cur=cur, c=c):
            vec = jnp.zeros((L,), jnp.float32)
            for r in range(L):
                i = g * L + r
                acc = ubufs[cur][i, pl.ds(0, L)] * nbufs[cur][i, pl.ds(0, L)]
                for j in range(1, D // L):
                    acc = acc + (ubufs[cur][i, pl.ds(j * L, L)]
                                 * nbufs[cur][i, pl.ds(j * L, L)])
                s = jnp.sum(acc)
                vec = jnp.where(lanes == r, s, vec)
            scores[pl.ds(c * CH + g * L, L)] = vec
            return 0

        lax.fori_loop(0, CH // L, grp_body, 0)

    pltpu.sync_copy(scores, out_ref.at[pl.ds(base, BPW)])


@jax.jit
def _scores(user, news, user_table, news_table):
    mesh = plsc.VectorSubcoreMesh(core_axis_name="c", subcore_axis_name="s",
                                  num_cores=NC, num_subcores=NS)
    call = functools.partial(
        pl.kernel,
        out_type=jax.ShapeDtypeStruct((B,), jnp.float32),
        mesh=mesh,
        compiler_params=pltpu.CompilerParams(needs_layout_passes=False),
        scratch_types=[
            pltpu.VMEM((NCHUNK, CH), jnp.int32),
            pltpu.VMEM((NCHUNK, CH), jnp.int32),
            pltpu.VMEM((CH, D), jnp.float32),
            pltpu.VMEM((CH, D), jnp.float32),
            pltpu.VMEM((CH, D), jnp.float32),
            pltpu.VMEM((CH, D), jnp.float32),
            pltpu.VMEM((BPW,), jnp.float32),
            pltpu.SemaphoreType.DMA,
            pltpu.SemaphoreType.DMA,
        ],
    )(_sc_body)
    return call(user.astype(jnp.int32), news.astype(jnp.int32),
                user_table, news_table)


def kernel(user, news, user_table, news_table):
    scores = _scores(user, news, user_table, news_table)
    one = lax.optimization_barrier(jnp.float32(1.0))
    ut = user_table * one
    nt = news_table * one
    return (ut, nt, scores)
